# baseline (speedup 1.0000x reference)
import functools

import jax
import jax.numpy as jnp
from jax.experimental import pallas as pl

S = 64.0
MARGIN = 0.4


def _cosface_block(labels_ref, x_ref, o_ref, *, block_cols):
    j = pl.program_id(1)
    lab = labels_ref[0, 0, :]
    x = x_ref[...]
    br, bc = x.shape
    col = jax.lax.broadcasted_iota(jnp.int32, (br, bc), 1) + j * block_cols
    hit = (col == lab[:, None]) & (lab[:, None] != -1)
    o_ref[...] = jnp.where(hit, (x - MARGIN) * S, x * S)


def kernel(logits, labels, embeddings):
    B, V = logits.shape
    block_rows = 256
    block_cols = 4096
    nrb = B // block_rows
    ncb = pl.cdiv(V, block_cols)
    labels32 = labels.astype(jnp.int32).reshape(nrb, 1, block_rows)

    out = pl.pallas_call(
        functools.partial(_cosface_block, block_cols=block_cols),
        grid=(nrb, ncb),
        in_specs=[
            pl.BlockSpec((1, 1, block_rows), lambda i, j: (i, 0, 0)),
            pl.BlockSpec((block_rows, block_cols), lambda i, j: (i, j)),
        ],
        out_specs=pl.BlockSpec((block_rows, block_cols), lambda i, j: (i, j)),
        out_shape=jax.ShapeDtypeStruct((B, V), logits.dtype),
    )(labels32, logits)
    return out


# speedup vs baseline: 1.0211x; 1.0211x over previous
import functools

import jax
import jax.numpy as jnp
from jax.experimental import pallas as pl

S = 64.0
MARGIN = 0.4


def _cosface_block(labels_ref, x_ref, o_ref):
    lab = labels_ref[0, 0, :]
    x = x_ref[...]
    br, bc = x.shape
    col = jax.lax.broadcasted_iota(jnp.int32, (br, bc), 1)
    hit = (col == lab[:, None]) & (lab[:, None] != -1)
    o_ref[...] = jnp.where(hit, (x - MARGIN) * S, x * S)


def kernel(logits, labels, embeddings):
    B, V = logits.shape
    block_rows = 16
    nrb = B // block_rows
    labels32 = labels.astype(jnp.int32).reshape(nrb, 1, block_rows)

    out = pl.pallas_call(
        _cosface_block,
        grid=(nrb,),
        in_specs=[
            pl.BlockSpec((1, 1, block_rows), lambda i: (i, 0, 0)),
            pl.BlockSpec((block_rows, V), lambda i: (i, 0)),
        ],
        out_specs=pl.BlockSpec((block_rows, V), lambda i: (i, 0)),
        out_shape=jax.ShapeDtypeStruct((B, V), logits.dtype),
    )(labels32, logits)
    return out
